# baseline (speedup 1.0000x reference)
import functools

import jax
import jax.numpy as jnp
from jax import lax
from jax.experimental import pallas as pl
from jax.experimental.pallas import tpu as pltpu
from jax.experimental.pallas import tpu_sc as plsc

_NC = 2
_NS = 16
_NW = _NC * _NS
_L = 16


@functools.cache
def _build_sc_fused(B, D, eps):
    rows_per_w = B // _NW
    CHUNK = 32
    n_chunks = rows_per_w // CHUNK
    POSR = CHUNK // 4
    n_vec = D // _L
    NBUF = 2

    mesh = plsc.VectorSubcoreMesh(core_axis_name="c", subcore_axis_name="s")

    @functools.partial(
        pl.kernel,
        out_type=jax.ShapeDtypeStruct((B, D), jnp.float32),
        mesh=mesh,
        scratch_types=[
            pltpu.VMEM((rows_per_w,), jnp.int32),
            pltpu.VMEM((rows_per_w + _L,), jnp.int32),
            pltpu.VMEM((NBUF, CHUNK, D), jnp.float32),
            pltpu.VMEM((NBUF, POSR, D), jnp.float32),
            pltpu.VMEM((2, D), jnp.float32),
            pltpu.VMEM((D,), jnp.float32),
            pltpu.VMEM((D,), jnp.float32),
            pltpu.VMEM((D,), jnp.float32),
            pltpu.SMEM((2 * CHUNK,), jnp.float32),
            pltpu.SemaphoreType.DMA((NBUF,)),
            pltpu.SemaphoreType.DMA((NBUF,)),
            pltpu.SemaphoreType.DMA((NBUF,)),
        ],
    )
    def sc_fused(src_ref, tt_ref, emb_ref, pos_ref, ttab_ref, g_ref, b_ref,
                 out_ref, idx_v, tts_v, x_buf, p_buf, ttab_v, td_v, g_v, b_v,
                 stats, gsems, psems, osems):
        wid = lax.axis_index("s") * _NC + lax.axis_index("c")
        base = wid * rows_per_w
        sbase = wid * (rows_per_w // 4)

        pltpu.sync_copy(src_ref.at[pl.ds(base, rows_per_w)], idx_v)
        pltpu.sync_copy(tt_ref.at[pl.ds(base, rows_per_w)],
                        tts_v.at[pl.ds(0, rows_per_w)])
        pltpu.sync_copy(ttab_ref, ttab_v)
        pltpu.sync_copy(g_ref, g_v)
        pltpu.sync_copy(b_ref, b_v)
        for j in range(n_vec):
            sl = pl.ds(j * _L, _L)
            td_v[sl] = ttab_v[1, sl] - ttab_v[0, sl]

        def start_in(c, b):
            pltpu.async_copy(
                emb_ref.at[idx_v.at[pl.ds(c * CHUNK, CHUNK)]],
                x_buf.at[b], gsems.at[b])
            s0 = pl.multiple_of(sbase + c * POSR, POSR)
            pltpu.async_copy(pos_ref.at[pl.ds(s0, POSR)], p_buf.at[b],
                             psems.at[b])

        def wait_in(c, b):
            pltpu.make_async_copy(
                emb_ref.at[idx_v.at[pl.ds(c * CHUNK, CHUNK)]],
                x_buf.at[b], gsems.at[b]).wait()
            s0 = pl.multiple_of(sbase + c * POSR, POSR)
            pltpu.make_async_copy(pos_ref.at[pl.ds(s0, POSR)], p_buf.at[b],
                                  psems.at[b]).wait()

        def out_slice(c):
            return out_ref.at[pl.ds(pl.multiple_of(base + c * CHUNK, CHUNK),
                                    CHUNK)]

        start_in(0, 0)

        def chunk_body(c, carry):
            b = lax.rem(c, NBUF)

            @pl.when(c + 1 < n_chunks)
            def _():
                b1 = lax.rem(c + 1, NBUF)

                @pl.when(c >= 1)
                def _():
                    pltpu.make_async_copy(x_buf.at[b1], out_slice(c - 1),
                                          osems.at[b1]).wait()

                start_in(c + 1, b1)

            wait_in(c, b)

            def posfold(j, pc):
                sl = pl.ds(j * _L, _L)
                t0 = ttab_v[0, sl]
                for r8 in range(POSR):
                    p_buf[b, r8, sl] = p_buf[b, r8, sl] + t0
                return pc

            lax.fori_loop(0, n_vec, posfold, 0)

            def row_body(r, rc):
                s8 = lax.shift_right_logical(r, 2)
                w = tts_v[pl.ds(c * CHUNK + r, _L)][0].astype(jnp.float32)
                a0 = jnp.zeros((_L,), jnp.float32)
                a1 = jnp.zeros((_L,), jnp.float32)
                q0 = jnp.zeros((_L,), jnp.float32)
                q1 = jnp.zeros((_L,), jnp.float32)
                for j in range(n_vec):
                    sl = pl.ds(j * _L, _L)
                    x = (x_buf[b, r, sl] + p_buf[b, s8, sl] + w * td_v[sl])
                    x_buf[b, r, sl] = x
                    if j % 2 == 0:
                        a0 = a0 + x
                        q0 = q0 + x * x
                    else:
                        a1 = a1 + x
                        q1 = q1 + x * x
                a0 = a0 + a1
                q0 = q0 + q1
                s1 = a0[0]
                s2 = q0[0]
                for i in range(1, _L):
                    s1 = s1 + a0[i]
                    s2 = s2 + q0[i]
                mean = s1 * (1.0 / D)
                var = s2 * (1.0 / D) - mean * mean
                vs = var + eps
                bi = 0x5F3759DF - lax.shift_right_logical(
                    lax.bitcast_convert_type(vs, jnp.int32), 1)
                g = lax.bitcast_convert_type(bi, jnp.float32)
                for _ in range(3):
                    g = g * (1.5 - 0.5 * vs * g * g)
                stats[2 * r] = mean
                stats[2 * r + 1] = g
                return rc

            lax.fori_loop(0, CHUNK, row_body, 0)

            def col_body(j, jc):
                sl = pl.ds(j * _L, _L)
                gj = g_v[sl]
                bj = b_v[sl]
                for r in range(CHUNK):
                    mean = stats[2 * r]
                    rstd = stats[2 * r + 1]
                    xv = x_buf[b, r, sl]
                    x_buf[b, r, sl] = (xv - mean) * rstd * gj + bj
                return jc

            lax.fori_loop(0, n_vec, col_body, 0)

            pltpu.async_copy(x_buf.at[b], out_slice(c), osems.at[b])
            return carry

        lax.fori_loop(0, n_chunks, chunk_body, 0)

        for c in (n_chunks - 2, n_chunks - 1):
            pltpu.make_async_copy(x_buf.at[c % NBUF], out_slice(c),
                                  osems.at[c % NBUF]).wait()

    return sc_fused


def kernel(src, token_type_input, embed_table, pos_table, tok_type_table,
           ln_gamma, ln_beta):
    S, N = src.shape
    D = embed_table.shape[1]
    B = S * N
    out = _build_sc_fused(B, D, 1e-5)(
        src.reshape(B).astype(jnp.int32),
        token_type_input.reshape(B).astype(jnp.int32),
        embed_table,
        pos_table,
        tok_type_table,
        ln_gamma,
        ln_beta,
    )
    return out.reshape(S, N, D)


# speedup vs baseline: 1.8621x; 1.8621x over previous
import functools

import jax
import jax.numpy as jnp
from jax import lax
from jax.experimental import pallas as pl
from jax.experimental.pallas import tpu as pltpu
from jax.experimental.pallas import tpu_sc as plsc

_NC = 2
_NS = 16
_NW = _NC * _NS


@functools.cache
def _build_sc_gather(B, D):
    rows_per_w = B // _NW
    CHUNK = 32
    n_chunks = rows_per_w // CHUNK
    NBUF = 3

    mesh = plsc.VectorSubcoreMesh(core_axis_name="c", subcore_axis_name="s")

    @functools.partial(
        pl.kernel,
        out_type=jax.ShapeDtypeStruct((B, D), jnp.float32),
        mesh=mesh,
        scratch_types=[
            pltpu.VMEM((n_chunks, CHUNK), jnp.int32),
            pltpu.VMEM((NBUF, CHUNK, D), jnp.float32),
            pltpu.SemaphoreType.DMA((NBUF,)),
            pltpu.SemaphoreType.DMA((NBUF,)),
        ],
    )
    def sc_gather(src_ref, emb_ref, out_ref, idx_v, x_buf, gsems, osems):
        wid = lax.axis_index("s") * _NC + lax.axis_index("c")
        base = wid * rows_per_w
        for c in range(n_chunks):
            pltpu.sync_copy(src_ref.at[pl.ds(base + c * CHUNK, CHUNK)],
                            idx_v.at[c])

        gdescs = [None] * NBUF
        odescs = [None] * NBUF
        for c in range(n_chunks + 1):
            if c < n_chunks:
                b = c % NBUF
                if c >= NBUF:
                    odescs[b].wait()
                gdescs[b] = pltpu.async_copy(
                    emb_ref.at[idx_v.at[c]],
                    x_buf.at[b], gsems.at[b])
            if c >= 1:
                p = (c - 1) % NBUF
                gdescs[p].wait()
                odescs[p] = pltpu.async_copy(
                    x_buf.at[p],
                    out_ref.at[pl.ds(base + (c - 1) * CHUNK, CHUNK)],
                    osems.at[p])
        for c in range(max(0, n_chunks - NBUF + 1), n_chunks):
            odescs[c % NBUF].wait()

    return sc_gather


@functools.cache
def _build_tc_ln(S, N, D, eps):
    SB = 64
    grid = (S // SB,)

    def tc_ln(tok_ref, tt_ref, pos_ref, ttab_ref, g_ref, b_ref, out_ref):
        x = tok_ref[...]
        x = x + pos_ref[...][:, None, :]
        w = tt_ref[...].astype(jnp.float32)[..., None]
        t0 = ttab_ref[0]
        t1 = ttab_ref[1]
        x = x + t0[None, None, :] + w * (t1 - t0)[None, None, :]
        mean = jnp.mean(x, axis=-1, keepdims=True)
        xc = x - mean
        var = jnp.mean(xc * xc, axis=-1, keepdims=True)
        out_ref[...] = (xc * lax.rsqrt(var + eps) * g_ref[0][None, None, :]
                        + b_ref[0][None, None, :])

    return pl.pallas_call(
        tc_ln,
        grid=grid,
        in_specs=[
            pl.BlockSpec((SB, N, D), lambda i: (i, 0, 0)),
            pl.BlockSpec((SB, N), lambda i: (i, 0)),
            pl.BlockSpec((SB, D), lambda i: (i, 0)),
            pl.BlockSpec((2, D), lambda i: (0, 0)),
            pl.BlockSpec((1, D), lambda i: (0, 0)),
            pl.BlockSpec((1, D), lambda i: (0, 0)),
        ],
        out_specs=pl.BlockSpec((SB, N, D), lambda i: (i, 0, 0)),
        out_shape=jax.ShapeDtypeStruct((S, N, D), jnp.float32),
    )


def kernel(src, token_type_input, embed_table, pos_table, tok_type_table,
           ln_gamma, ln_beta):
    S, N = src.shape
    D = embed_table.shape[1]
    B = S * N
    tok = _build_sc_gather(B, D)(src.reshape(B).astype(jnp.int32),
                                 embed_table)
    out = _build_tc_ln(S, N, D, 1e-5)(
        tok.reshape(S, N, D),
        token_type_input.astype(jnp.int32),
        pos_table,
        tok_type_table,
        ln_gamma.reshape(1, D),
        ln_beta.reshape(1, D),
    )
    return out


# speedup vs baseline: 2.0180x; 1.0837x over previous
import functools

import jax
import jax.numpy as jnp
from jax import lax
from jax.experimental import pallas as pl
from jax.experimental.pallas import tpu as pltpu
from jax.experimental.pallas import tpu_sc as plsc

_NC = 2
_NS = 16
_NW = _NC * _NS


@functools.cache
def _build_sc_gather(B, D):
    rows_per_w = B // _NW
    CHUNK = 32
    n_chunks = rows_per_w // CHUNK
    NBUF = 3

    mesh = plsc.VectorSubcoreMesh(core_axis_name="c", subcore_axis_name="s")

    @functools.partial(
        pl.kernel,
        out_type=jax.ShapeDtypeStruct((B, D), jnp.float32),
        mesh=mesh,
        scratch_types=[
            pltpu.VMEM((rows_per_w,), jnp.int32),
            pltpu.VMEM((NBUF, CHUNK, D), jnp.float32),
            pltpu.SemaphoreType.DMA((NBUF,)),
            pltpu.SemaphoreType.DMA((NBUF,)),
        ],
    )
    def sc_gather(src_ref, emb_ref, out_ref, idx_v, x_buf, gsems, osems):
        wid = lax.axis_index("s") * _NC + lax.axis_index("c")
        base = wid * rows_per_w
        pltpu.sync_copy(src_ref.at[pl.ds(base, rows_per_w)], idx_v)

        gdescs = [None] * NBUF
        odescs = [None] * NBUF
        for c in range(n_chunks + 1):
            if c < n_chunks:
                b = c % NBUF
                if c >= NBUF:
                    odescs[b].wait()
                gdescs[b] = pltpu.async_copy(
                    emb_ref.at[idx_v.at[pl.ds(c * CHUNK, CHUNK)]],
                    x_buf.at[b], gsems.at[b])
            if c >= 1:
                p = (c - 1) % NBUF
                gdescs[p].wait()
                odescs[p] = pltpu.async_copy(
                    x_buf.at[p],
                    out_ref.at[pl.ds(base + (c - 1) * CHUNK, CHUNK)],
                    osems.at[p])
        for c in range(max(0, n_chunks - NBUF + 1), n_chunks):
            odescs[c % NBUF].wait()

    return sc_gather


@functools.cache
def _build_tc_ln(S, N, D, eps):
    SB = 128
    grid = (S // SB,)

    def tc_ln(tok_ref, tt_ref, pos_ref, ttab_ref, g_ref, b_ref, out_ref):
        x = tok_ref[...]
        x = x + pos_ref[...][:, None, :]
        w = tt_ref[...].astype(jnp.float32)[..., None]
        t0 = ttab_ref[0]
        t1 = ttab_ref[1]
        x = x + t0[None, None, :] + w * (t1 - t0)[None, None, :]
        mean = jnp.mean(x, axis=-1, keepdims=True)
        xc = x - mean
        var = jnp.mean(xc * xc, axis=-1, keepdims=True)
        out_ref[...] = (xc * lax.rsqrt(var + eps) * g_ref[0][None, None, :]
                        + b_ref[0][None, None, :])

    return pl.pallas_call(
        tc_ln,
        grid=grid,
        in_specs=[
            pl.BlockSpec((SB, N, D), lambda i: (i, 0, 0)),
            pl.BlockSpec((SB, N), lambda i: (i, 0)),
            pl.BlockSpec((SB, D), lambda i: (i, 0)),
            pl.BlockSpec((2, D), lambda i: (0, 0)),
            pl.BlockSpec((1, D), lambda i: (0, 0)),
            pl.BlockSpec((1, D), lambda i: (0, 0)),
        ],
        out_specs=pl.BlockSpec((SB, N, D), lambda i: (i, 0, 0)),
        out_shape=jax.ShapeDtypeStruct((S, N, D), jnp.float32),
    )


def kernel(src, token_type_input, embed_table, pos_table, tok_type_table,
           ln_gamma, ln_beta):
    S, N = src.shape
    D = embed_table.shape[1]
    B = S * N
    tok = _build_sc_gather(B, D)(src.reshape(B).astype(jnp.int32),
                                 embed_table)
    out = _build_tc_ln(S, N, D, 1e-5)(
        tok.reshape(S, N, D),
        token_type_input.astype(jnp.int32),
        pos_table,
        tok_type_table,
        ln_gamma.reshape(1, D),
        ln_beta.reshape(1, D),
    )
    return out


# speedup vs baseline: 2.0682x; 1.0249x over previous
import functools

import jax
import jax.numpy as jnp
from jax import lax
from jax.experimental import pallas as pl
from jax.experimental.pallas import tpu as pltpu
from jax.experimental.pallas import tpu_sc as plsc

_NC = 2
_NS = 16
_NW = _NC * _NS


@functools.cache
def _build_sc_gather(B, D):
    rows_per_w = B // _NW
    CHUNK = 32
    n_chunks = rows_per_w // CHUNK
    NBUF = 3

    mesh = plsc.VectorSubcoreMesh(core_axis_name="c", subcore_axis_name="s")

    @functools.partial(
        pl.kernel,
        out_type=jax.ShapeDtypeStruct((B, D), jnp.float32),
        mesh=mesh,
        scratch_types=[
            pltpu.VMEM((rows_per_w,), jnp.int32),
            pltpu.VMEM((NBUF, CHUNK, D), jnp.float32),
            pltpu.SemaphoreType.DMA((NBUF,)),
            pltpu.SemaphoreType.DMA((NBUF,)),
        ],
    )
    def sc_gather(src_ref, emb_ref, out_ref, idx_v, x_buf, gsems, osems):
        wid = lax.axis_index("s") * _NC + lax.axis_index("c")
        base = wid * rows_per_w
        pltpu.sync_copy(src_ref.at[pl.ds(base, rows_per_w)], idx_v)

        gdescs = [None] * NBUF
        odescs = [None] * NBUF
        for c in range(n_chunks + 1):
            if c < n_chunks:
                b = c % NBUF
                if c >= NBUF:
                    odescs[b].wait()
                gdescs[b] = pltpu.async_copy(
                    emb_ref.at[idx_v.at[pl.ds(c * CHUNK, CHUNK)]],
                    x_buf.at[b], gsems.at[b])
            if c >= 1:
                p = (c - 1) % NBUF
                gdescs[p].wait()
                odescs[p] = pltpu.async_copy(
                    x_buf.at[p],
                    out_ref.at[pl.ds(base + (c - 1) * CHUNK, CHUNK)],
                    osems.at[p])
        for c in range(max(0, n_chunks - NBUF + 1), n_chunks):
            odescs[c % NBUF].wait()

    return sc_gather


@functools.cache
def _build_tc_ln(S, N, D, eps):
    SB = 256
    grid = (S // SB,)

    def tc_ln(tok_ref, tt_ref, pos_ref, ttab_ref, g_ref, b_ref, out_ref):
        x = tok_ref[...]
        x = x + pos_ref[...][:, None, :]
        w = tt_ref[...].astype(jnp.float32)[..., None]
        t0 = ttab_ref[0]
        t1 = ttab_ref[1]
        x = x + t0[None, None, :] + w * (t1 - t0)[None, None, :]
        mean = jnp.mean(x, axis=-1, keepdims=True)
        xc = x - mean
        var = jnp.mean(xc * xc, axis=-1, keepdims=True)
        out_ref[...] = (xc * lax.rsqrt(var + eps) * g_ref[0][None, None, :]
                        + b_ref[0][None, None, :])

    return pl.pallas_call(
        tc_ln,
        grid=grid,
        in_specs=[
            pl.BlockSpec((SB, N, D), lambda i: (i, 0, 0)),
            pl.BlockSpec((SB, N), lambda i: (i, 0)),
            pl.BlockSpec((SB, D), lambda i: (i, 0)),
            pl.BlockSpec((2, D), lambda i: (0, 0)),
            pl.BlockSpec((1, D), lambda i: (0, 0)),
            pl.BlockSpec((1, D), lambda i: (0, 0)),
        ],
        out_specs=pl.BlockSpec((SB, N, D), lambda i: (i, 0, 0)),
        out_shape=jax.ShapeDtypeStruct((S, N, D), jnp.float32),
    )


def kernel(src, token_type_input, embed_table, pos_table, tok_type_table,
           ln_gamma, ln_beta):
    S, N = src.shape
    D = embed_table.shape[1]
    B = S * N
    tok = _build_sc_gather(B, D)(src.reshape(B).astype(jnp.int32),
                                 embed_table)
    out = _build_tc_ln(S, N, D, 1e-5)(
        tok.reshape(S, N, D),
        token_type_input.astype(jnp.int32),
        pos_table,
        tok_type_table,
        ln_gamma.reshape(1, D),
        ln_beta.reshape(1, D),
    )
    return out
